# baseline (speedup 1.0000x reference)
import functools
import numpy as np
import jax
import jax.numpy as jnp
from jax import lax
from jax.experimental import pallas as pl
from jax.experimental.pallas import tpu as pltpu, tpu_sc as plsc

N_NODES = 25000
E_EDGES = 400000
T_EDGES = 200000
HEADS = 4
DH = 32
MMBLK = 1000

SC_NC = 2
SC_NS = 16
SC_NW = SC_NC * SC_NS
GCH = 640
N_CHUNKS = E_EDGES // GCH
CH_PER_W = -(-N_CHUNKS // SC_NW)


def _gather3_body(tk, ik, tq, iq, tv, iv, kg, qg, vg, idx_v, rows_v, sem):
    w = lax.axis_index("s") * SC_NC + lax.axis_index("c")

    def chunk(j, carry):
        cw = w + j * SC_NW

        @pl.when(cw < N_CHUNKS)
        def _():
            off = pl.multiple_of(cw * GCH, GCH)
            for tab, ind, out in ((tk, ik, kg), (tq, iq, qg), (tv, iv, vg)):
                pltpu.sync_copy(ind.at[pl.ds(off, GCH)], idx_v)
                pltpu.async_copy(tab.at[idx_v], rows_v, sem).wait()
                pltpu.sync_copy(rows_v, out.at[pl.ds(off, GCH)])
        return carry

    lax.fori_loop(0, CH_PER_W, chunk, 0)


def _gather3(tk, ik, tq, iq, tv, iv):
    fn = pl.kernel(
        _gather3_body,
        out_type=[jax.ShapeDtypeStruct((E_EDGES, 128), jnp.float32)] * 3,
        mesh=plsc.VectorSubcoreMesh(core_axis_name="c", subcore_axis_name="s"),
        scratch_types=[
            pltpu.VMEM((GCH,), jnp.int32),
            pltpu.VMEM((GCH, 128), jnp.float32),
            pltpu.SemaphoreType.DMA,
        ],
    )
    return fn(tk, ik, tq, iq, tv, iv)


NPAD = 25088
ACCR = NPAD // 2
SROWS = ACCR // SC_NS
SCCH = 224
EPT = E_EDGES // SC_NS
NCH = EPT // SCCH
TAIL = EPT - NCH * SCCH


def _scat_body(msg, dst2, out, acc, idx_v, rows_v, sem):
    c = lax.axis_index("c")
    s = lax.axis_index("s")
    r0 = s * SROWS

    def do_chunk(h, eoff, n):
        pltpu.sync_copy(dst2.at[pl.ds(eoff, n)], idx_v.at[pl.ds(0, n)])
        pltpu.sync_copy(msg.at[pl.ds(h * E_EDGES + eoff, n)],
                        rows_v.at[pl.ds(0, n)])
        if n < SCCH:
            def ztail(i, carry):
                for j in range(8):
                    rows_v[n + i, pl.ds(j * 16, 16)] = (
                        jnp.zeros((16,), jnp.float32))
                return carry
            lax.fori_loop(0, SCCH - n, ztail, 0)
        pltpu.sync_copy(rows_v, acc.at[idx_v], add=True)

    for hh in range(2):
        h = c * 2 + hh

        def zrow(i, carry):
            for j in range(8):
                rows_v[i, pl.ds(j * 16, 16)] = jnp.zeros((16,), jnp.float32)
            return carry

        lax.fori_loop(0, SCCH, zrow, 0)
        for k in range(3):
            pltpu.sync_copy(rows_v, acc.at[pl.ds(r0 + k * SCCH, SCCH)])
        pltpu.sync_copy(rows_v.at[pl.ds(0, SROWS - 3 * SCCH)],
                        acc.at[pl.ds(r0 + 3 * SCCH, SROWS - 3 * SCCH)])
        plsc.subcore_barrier()

        def chunk(j, carry):
            do_chunk(h, s * EPT + j * SCCH, SCCH)
            return carry

        lax.fori_loop(0, NCH, chunk, 0)
        do_chunk(h, s * EPT + NCH * SCCH, TAIL)
        plsc.subcore_barrier()
        pltpu.sync_copy(acc.at[pl.ds(r0, SROWS)], out.at[h, pl.ds(r0, SROWS)])
        plsc.subcore_barrier()


def _scatter_nd(msg128, dst2):
    fn = pl.kernel(
        _scat_body,
        out_type=jax.ShapeDtypeStruct((HEADS, ACCR, 128), jnp.float32),
        mesh=plsc.VectorSubcoreMesh(core_axis_name="c", subcore_axis_name="s"),
        scratch_types=[
            pltpu.VMEM_SHARED((ACCR, 128), jnp.float32),
            pltpu.VMEM((SCCH,), jnp.int32),
            pltpu.VMEM((SCCH, 128), jnp.float32),
            pltpu.SemaphoreType.DMA,
        ],
    )
    return fn(msg128, dst2)


def _alpha_body(kg_ref, qg_ref, o_ref):
    p = kg_ref[...] * qg_ref[...]
    o_ref[...] = p.reshape(-1, HEADS, DH).sum(-1)


def _alpha_tc(kg, qg, blk=2000):
    m = kg.shape[0]
    return pl.pallas_call(
        _alpha_body,
        grid=(m // blk,),
        in_specs=[pl.BlockSpec((blk, 128), lambda i: (i, 0)),
                  pl.BlockSpec((blk, 128), lambda i: (i, 0))],
        out_specs=pl.BlockSpec((blk, HEADS), lambda i: (i, 0)),
        out_shape=jax.ShapeDtypeStruct((m, HEADS), jnp.float32),
    )(kg, qg)


def _msg128_body(vg_ref, ex_ref, par_ref, r_ref, s_ref, e_ref,
                 b0_ref, b1_ref, o_ref):
    exrep = jnp.dot(ex_ref[...], r_ref[0],
                    preferred_element_type=jnp.float32)
    half = (jnp.dot(vg_ref[...] * exrep, s_ref[0],
                    preferred_element_type=jnp.float32)
            + jnp.dot(ex_ref[...], e_ref[0],
                      preferred_element_type=jnp.float32))
    p = par_ref[...]
    o_ref[...] = (jnp.dot(half, b0_ref[...],
                          preferred_element_type=jnp.float32) * (1.0 - p)
                  + jnp.dot(half, b1_ref[...],
                            preferred_element_type=jnp.float32) * p)


def _sel_consts():
    r = np.zeros((HEADS, HEADS, 128), np.float32)
    s = np.zeros((HEADS, 128, 64), np.float32)
    e = np.zeros((HEADS, HEADS, 64), np.float32)
    for h in range(HEADS):
        for c in range(DH):
            r[h, h, h * DH + c] = 1.0
            s[h, h * DH + c, c] = 1.0
        e[h, h, DH] = 1.0
    b0 = np.zeros((64, 128), np.float32)
    b1 = np.zeros((64, 128), np.float32)
    for c in range(64):
        b0[c, c] = 1.0
        b1[c, 64 + c] = 1.0
    return (jnp.asarray(r), jnp.asarray(s), jnp.asarray(e),
            jnp.asarray(b0), jnp.asarray(b1))


def _msg128_tc(vg, ex, par, blk=2000):
    m = vg.shape[0]
    rc, sc, ec, b0c, b1c = _sel_consts()
    return pl.pallas_call(
        _msg128_body,
        grid=(HEADS, m // blk),
        in_specs=[pl.BlockSpec((blk, 128), lambda h, i: (i, 0)),
                  pl.BlockSpec((blk, HEADS), lambda h, i: (i, 0)),
                  pl.BlockSpec((blk, 1), lambda h, i: (i, 0)),
                  pl.BlockSpec((1, HEADS, 128), lambda h, i: (h, 0, 0)),
                  pl.BlockSpec((1, 128, 64), lambda h, i: (h, 0, 0)),
                  pl.BlockSpec((1, HEADS, 64), lambda h, i: (h, 0, 0)),
                  pl.BlockSpec((64, 128), lambda h, i: (0, 0)),
                  pl.BlockSpec((64, 128), lambda h, i: (0, 0))],
        out_specs=pl.BlockSpec((blk, 128),
                               lambda h, i: (h * (m // blk) + i, 0)),
        out_shape=jax.ShapeDtypeStruct((HEADS * m, 128), jnp.float32),
    )(vg, ex, par, rc, sc, ec, b0c, b1c)


def _mm_body(x_ref, w_ref, b_ref, o_ref, *, act):
    acc = jnp.dot(x_ref[...], w_ref[...], preferred_element_type=jnp.float32)
    acc = acc + b_ref[...]
    if act == "relu":
        acc = jnp.maximum(acc, 0.0)
    o_ref[...] = acc


def _mm(x, w, b, act="none", blk=MMBLK):
    m, kin = x.shape
    kout = w.shape[1]
    assert m % blk == 0
    grid = (m // blk,)
    return pl.pallas_call(
        functools.partial(_mm_body, act=act),
        grid=grid,
        in_specs=[
            pl.BlockSpec((blk, kin), lambda i: (i, 0)),
            pl.BlockSpec((kin, kout), lambda i: (0, 0)),
            pl.BlockSpec((1, kout), lambda i: (0, 0)),
        ],
        out_specs=pl.BlockSpec((blk, kout), lambda i: (i, 0)),
        out_shape=jax.ShapeDtypeStruct((m, kout), jnp.float32),
    )(x, w, b.reshape(1, kout))


def _gelu(x):
    return 0.5 * x * (1.0 + jax.lax.erf(x * np.float32(1.0 / np.sqrt(2.0))))


def _post_body(agg_ref, h_ref, wa_ref, ba_ref, g_ref, o_ref, *, act):
    g = _gelu(agg_ref[...])
    o = jnp.dot(g, wa_ref[...], preferred_element_type=jnp.float32)
    o = o + ba_ref[...] + g_ref[...] * h_ref[...]
    if act == "relu":
        o = jnp.maximum(o, 0.0)
    o_ref[...] = o


def _post(agg, h, wa, ba, gamma, act="none", blk=MMBLK):
    m, k = agg.shape
    grid = (m // blk,)
    return pl.pallas_call(
        functools.partial(_post_body, act=act),
        grid=grid,
        in_specs=[
            pl.BlockSpec((blk, k), lambda i: (i, 0)),
            pl.BlockSpec((blk, k), lambda i: (i, 0)),
            pl.BlockSpec((k, k), lambda i: (0, 0)),
            pl.BlockSpec((1, k), lambda i: (0, 0)),
            pl.BlockSpec((1, 1), lambda i: (0, 0)),
        ],
        out_specs=pl.BlockSpec((blk, k), lambda i: (i, 0)),
        out_shape=jax.ShapeDtypeStruct((m, k), jnp.float32),
    )(agg, h, wa, ba.reshape(1, k), gamma.reshape(1, 1))


def _dec_body(pg_ref, qg_ref, at_ref, w1c_ref, b1_ref, w2_ref, b2_ref, o_ref):
    s = pg_ref[...] + qg_ref[...] + b1_ref[...]
    s = s + jnp.dot(at_ref[...], w1c_ref[...], preferred_element_type=jnp.float32)
    s = jnp.maximum(s, 0.0)
    o_ref[...] = (jnp.dot(s, w2_ref[...], preferred_element_type=jnp.float32)
                  + b2_ref[...])


def _dec_final(pg, qg, attr, w1c, b1, w2, b2, blk=MMBLK):
    m, k = pg.shape
    ea = attr.shape[1]
    grid = (m // blk,)
    return pl.pallas_call(
        _dec_body,
        grid=grid,
        in_specs=[
            pl.BlockSpec((blk, k), lambda i: (i, 0)),
            pl.BlockSpec((blk, k), lambda i: (i, 0)),
            pl.BlockSpec((blk, ea), lambda i: (i, 0)),
            pl.BlockSpec((ea, k), lambda i: (0, 0)),
            pl.BlockSpec((1, k), lambda i: (0, 0)),
            pl.BlockSpec((k, 1), lambda i: (0, 0)),
            pl.BlockSpec((1, 1), lambda i: (0, 0)),
        ],
        out_specs=pl.BlockSpec((blk, 1), lambda i: (i, 0)),
        out_shape=jax.ShapeDtypeStruct((m, 1), jnp.float32),
    )(pg, qg, attr, w1c, b1.reshape(1, k), w2, b2.reshape(1, 1))


def _fold_rel(w, b, rel, scale=None):
    wf = jnp.einsum("ihd,hde->ihe", w.reshape(128, HEADS, DH), rel)
    bf = jnp.einsum("hd,hde->he", b.reshape(HEADS, DH), rel)
    if scale is not None:
        wf = wf * scale[None, :, None]
        bf = bf * scale[:, None]
    return wf.reshape(128, 128), bf.reshape(128)


def _layer_weights(params, c):
    out = {}
    rel_of_src = {"product": "pw", "warehouse": "wp"}
    for t in ("product", "warehouse"):
        r = rel_of_src[t]
        scale = params[c + "_prel_" + r] * np.float32(1.0 / np.sqrt(DH))
        wk, bk = _fold_rel(params[c + "_k_" + t + "_w"],
                           params[c + "_k_" + t + "_b"],
                           params[c + "_arel_" + r], scale)
        wv, bv = _fold_rel(params[c + "_v_" + t + "_w"],
                           params[c + "_v_" + t + "_b"],
                           params[c + "_mrel_" + r])
        wcat = jnp.concatenate(
            [params[c + "_q_" + t + "_w"], wk, wv], axis=1)
        bcat = jnp.concatenate(
            [params[c + "_q_" + t + "_b"], bk, bv], axis=0)
        out[t] = (wcat, bcat)
    return out


def _edge_pass(k_rel_s, q_d, v_rel_s, src, dst):
    kg, qg, vg = _gather3(k_rel_s, src, q_d, dst, v_rel_s, src)
    alpha = _alpha_tc(kg, qg)
    amax = jax.ops.segment_max(alpha, dst, num_segments=N_NODES)
    amax = jnp.where(jnp.isfinite(amax), amax, 0.0)
    ex = jnp.exp(alpha - amax[dst])
    par = (dst & 1).astype(jnp.float32)[:, None]
    nd = _scatter_nd(_msg128_tc(vg, ex, par), dst >> 1)
    flat = nd.reshape(HEADS, NPAD, 64)[:, :N_NODES, :]
    num = flat[:, :, :DH].transpose(1, 0, 2).reshape(N_NODES, 128)
    den = flat[:, :, DH].T
    return num / (den + 1e-16).repeat(DH, axis=1)


def _hgt_layer(c, h, e_pw, e_wp, params):
    lw = _layer_weights(params, c)
    proj = {}
    for t in ("product", "warehouse"):
        w, b = lw[t]
        z = _mm(h[t], w, b)
        proj[t] = (z[:, :128], z[:, 128:256], z[:, 256:384])
    agg = {}
    for r, s, d, ei in (("pw", "product", "warehouse", e_pw),
                        ("wp", "warehouse", "product", e_wp)):
        agg[d] = _edge_pass(proj[s][1], proj[d][0], proj[s][2], ei[0], ei[1])
    out = {}
    for t in ("product", "warehouse"):
        beta = jax.nn.sigmoid(params[c + "_skip_" + t])
        wa = params[c + "_a_" + t + "_w"] * beta
        ba = params[c + "_a_" + t + "_b"] * beta
        out[t] = _post(agg[t], h[t], wa, ba, 1.0 - beta,
                       act="relu" if c == "c1" else "none")
    return out


def kernel(x_product, x_warehouse, edge_index_pw, edge_index_wp,
           target_edge_index, target_edge_attr, params):
    h = {
        "product": _mm(x_product, params["in_product_w"],
                       params["in_product_b"], act="relu"),
        "warehouse": _mm(x_warehouse, params["in_warehouse_w"],
                         params["in_warehouse_b"], act="relu"),
    }
    h = _hgt_layer("c1", h, edge_index_pw, edge_index_wp, params)
    h = _hgt_layer("c2", h, edge_index_pw, edge_index_wp, params)

    w1 = params["dec1_w"]
    zeros = jnp.zeros((128,), jnp.float32)
    p_arr = _mm(h["product"], w1[:128], zeros)
    q_arr = _mm(h["warehouse"], w1[128:256], zeros)
    src, dst = target_edge_index[0], target_edge_index[1]
    out = _dec_final(p_arr[src], q_arr[dst], target_edge_attr,
                     w1[256:260], params["dec1_b"],
                     params["dec2_w"], params["dec2_b"])
    return out.reshape(-1)


# speedup vs baseline: 1.0206x; 1.0206x over previous
import functools
import numpy as np
import jax
import jax.numpy as jnp
from jax import lax
from jax.experimental import pallas as pl
from jax.experimental.pallas import tpu as pltpu, tpu_sc as plsc

N_NODES = 25000
E_EDGES = 400000
T_EDGES = 200000
HEADS = 4
DH = 32
MMBLK = 1000

SC_NC = 2
SC_NS = 16
SC_NW = SC_NC * SC_NS
GCH = 640
N_CHUNKS = E_EDGES // GCH
CH_PER_W = -(-N_CHUNKS // SC_NW)


def _gather3_body(tk, ik, tq, iq, tv, iv, kg, qg, vg, idx_v, rows_v, sem):
    w = lax.axis_index("s") * SC_NC + lax.axis_index("c")

    def chunk(j, carry):
        cw = w + j * SC_NW

        @pl.when(cw < N_CHUNKS)
        def _():
            off = pl.multiple_of(cw * GCH, GCH)
            for tab, ind, out in ((tk, ik, kg), (tq, iq, qg), (tv, iv, vg)):
                pltpu.sync_copy(ind.at[pl.ds(off, GCH)], idx_v)
                pltpu.async_copy(tab.at[idx_v], rows_v, sem).wait()
                pltpu.sync_copy(rows_v, out.at[pl.ds(off, GCH)])
        return carry

    lax.fori_loop(0, CH_PER_W, chunk, 0)


def _gather3(tk, ik, tq, iq, tv, iv):
    fn = pl.kernel(
        _gather3_body,
        out_type=[jax.ShapeDtypeStruct((E_EDGES, 128), jnp.float32)] * 3,
        mesh=plsc.VectorSubcoreMesh(core_axis_name="c", subcore_axis_name="s"),
        scratch_types=[
            pltpu.VMEM((GCH,), jnp.int32),
            pltpu.VMEM((GCH, 128), jnp.float32),
            pltpu.SemaphoreType.DMA,
        ],
    )
    return fn(tk, ik, tq, iq, tv, iv)


ACCR = 12544
NPAD = 2 * ACCR
SROWS = ACCR // SC_NS
SCCH = 112
EPT = E_EDGES // SC_NS
NCH = EPT // SCCH
TAIL = EPT - NCH * SCCH


def _scat_body(msg, dst2, out, acc, idx0, rows0, idx1, rows1, sl, ss):
    c = lax.axis_index("c")
    s = lax.axis_index("s")
    r0 = s * SROWS
    bufs = ((idx0, rows0), (idx1, rows1))

    def load(h, j, n, idx_v, rows_v):
        eoff = s * EPT + j * SCCH
        pltpu.async_copy(dst2.at[pl.ds(eoff, n)], idx_v.at[pl.ds(0, n)], sl)
        pltpu.async_copy(msg.at[pl.ds(h * E_EDGES + eoff, n)],
                         rows_v.at[pl.ds(0, n)], sl)

    def wait_load(h, j, n, idx_v, rows_v):
        eoff = s * EPT + j * SCCH
        pltpu.make_async_copy(dst2.at[pl.ds(eoff, n)],
                              idx_v.at[pl.ds(0, n)], sl).wait()
        pltpu.make_async_copy(msg.at[pl.ds(h * E_EDGES + eoff, n)],
                              rows_v.at[pl.ds(0, n)], sl).wait()

    def wait_scat(rows_v, idx_v):
        pltpu.make_async_copy(rows_v, acc.at[idx_v], ss).wait()

    def ztail(rows_v, n):
        def zr(i, carry):
            for j in range(8):
                rows_v[n + i, pl.ds(j * 16, 16)] = (
                    jnp.zeros((16,), jnp.float32))
            return carry
        lax.fori_loop(0, SCCH - n, zr, 0)

    for hh in range(2):
        h = c * 2 + hh

        def zrow(i, carry):
            for j in range(8):
                rows0[i, pl.ds(j * 16, 16)] = jnp.zeros((16,), jnp.float32)
            return carry

        lax.fori_loop(0, SCCH, zrow, 0)
        for k in range(SROWS // SCCH):
            pltpu.sync_copy(rows0, acc.at[pl.ds(r0 + k * SCCH, SCCH)])
        rem = SROWS % SCCH
        if rem:
            pltpu.sync_copy(rows0.at[pl.ds(0, rem)],
                            acc.at[pl.ds(r0 + SROWS - rem, rem)])
        plsc.subcore_barrier()

        load(h, 0, SCCH, *bufs[0])

        def pair(j2, carry):
            a = 2 * j2
            wait_load(h, a, SCCH, *bufs[0])

            @pl.when(a > 0)
            def _():
                wait_scat(rows1, idx1)

            @pl.when(a + 1 < NCH)
            def _():
                load(h, a + 1, SCCH, *bufs[1])
            pltpu.async_copy(rows0, acc.at[idx0], ss, add=True)

            @pl.when(a + 1 < NCH)
            def _():
                wait_load(h, a + 1, SCCH, *bufs[1])
                wait_scat(rows0, idx0)

                @pl.when(a + 2 < NCH)
                def _():
                    load(h, a + 2, SCCH, *bufs[0])
                pltpu.async_copy(rows1, acc.at[idx1], ss, add=True)
            return carry

        lax.fori_loop(0, (NCH + 1) // 2, pair, 0)
        wait_scat(rows0, idx0)

        eoff = s * EPT + NCH * SCCH
        pltpu.sync_copy(dst2.at[pl.ds(eoff, TAIL)], idx0.at[pl.ds(0, TAIL)])
        pltpu.sync_copy(msg.at[pl.ds(h * E_EDGES + eoff, TAIL)],
                        rows0.at[pl.ds(0, TAIL)])
        ztail(rows0, TAIL)
        pltpu.sync_copy(rows0, acc.at[idx0], add=True)

        plsc.subcore_barrier()
        pltpu.sync_copy(acc.at[pl.ds(r0, SROWS)], out.at[h, pl.ds(r0, SROWS)])
        plsc.subcore_barrier()


def _scatter_nd(msg128, dst2):
    fn = pl.kernel(
        _scat_body,
        out_type=jax.ShapeDtypeStruct((HEADS, ACCR, 128), jnp.float32),
        mesh=plsc.VectorSubcoreMesh(core_axis_name="c", subcore_axis_name="s"),
        scratch_types=[
            pltpu.VMEM_SHARED((ACCR, 128), jnp.float32),
            pltpu.VMEM((SCCH,), jnp.int32),
            pltpu.VMEM((SCCH, 128), jnp.float32),
            pltpu.VMEM((SCCH,), jnp.int32),
            pltpu.VMEM((SCCH, 128), jnp.float32),
            pltpu.SemaphoreType.DMA,
            pltpu.SemaphoreType.DMA,
        ],
    )
    return fn(msg128, dst2)


def _alpha_body(kg_ref, qg_ref, o_ref):
    p = kg_ref[...] * qg_ref[...]
    o_ref[...] = p.reshape(-1, HEADS, DH).sum(-1)


def _alpha_tc(kg, qg, blk=2000):
    m = kg.shape[0]
    return pl.pallas_call(
        _alpha_body,
        grid=(m // blk,),
        in_specs=[pl.BlockSpec((blk, 128), lambda i: (i, 0)),
                  pl.BlockSpec((blk, 128), lambda i: (i, 0))],
        out_specs=pl.BlockSpec((blk, HEADS), lambda i: (i, 0)),
        out_shape=jax.ShapeDtypeStruct((m, HEADS), jnp.float32),
    )(kg, qg)


def _msg128_body(vg_ref, ex_ref, par_ref, r_ref, s_ref, e_ref,
                 b0_ref, b1_ref, o_ref):
    exrep = jnp.dot(ex_ref[...], r_ref[0],
                    preferred_element_type=jnp.float32)
    half = (jnp.dot(vg_ref[...] * exrep, s_ref[0],
                    preferred_element_type=jnp.float32)
            + jnp.dot(ex_ref[...], e_ref[0],
                      preferred_element_type=jnp.float32))
    p = par_ref[...]
    o_ref[...] = (jnp.dot(half, b0_ref[...],
                          preferred_element_type=jnp.float32) * (1.0 - p)
                  + jnp.dot(half, b1_ref[...],
                            preferred_element_type=jnp.float32) * p)


def _sel_consts():
    r = np.zeros((HEADS, HEADS, 128), np.float32)
    s = np.zeros((HEADS, 128, 64), np.float32)
    e = np.zeros((HEADS, HEADS, 64), np.float32)
    for h in range(HEADS):
        for c in range(DH):
            r[h, h, h * DH + c] = 1.0
            s[h, h * DH + c, c] = 1.0
        e[h, h, DH] = 1.0
    b0 = np.zeros((64, 128), np.float32)
    b1 = np.zeros((64, 128), np.float32)
    for c in range(64):
        b0[c, c] = 1.0
        b1[c, 64 + c] = 1.0
    return (jnp.asarray(r), jnp.asarray(s), jnp.asarray(e),
            jnp.asarray(b0), jnp.asarray(b1))


def _msg128_tc(vg, ex, par, blk=2000):
    m = vg.shape[0]
    rc, sc, ec, b0c, b1c = _sel_consts()
    return pl.pallas_call(
        _msg128_body,
        grid=(HEADS, m // blk),
        in_specs=[pl.BlockSpec((blk, 128), lambda h, i: (i, 0)),
                  pl.BlockSpec((blk, HEADS), lambda h, i: (i, 0)),
                  pl.BlockSpec((blk, 1), lambda h, i: (i, 0)),
                  pl.BlockSpec((1, HEADS, 128), lambda h, i: (h, 0, 0)),
                  pl.BlockSpec((1, 128, 64), lambda h, i: (h, 0, 0)),
                  pl.BlockSpec((1, HEADS, 64), lambda h, i: (h, 0, 0)),
                  pl.BlockSpec((64, 128), lambda h, i: (0, 0)),
                  pl.BlockSpec((64, 128), lambda h, i: (0, 0))],
        out_specs=pl.BlockSpec((blk, 128),
                               lambda h, i: (h * (m // blk) + i, 0)),
        out_shape=jax.ShapeDtypeStruct((HEADS * m, 128), jnp.float32),
    )(vg, ex, par, rc, sc, ec, b0c, b1c)


def _mm_body(x_ref, w_ref, b_ref, o_ref, *, act):
    acc = jnp.dot(x_ref[...], w_ref[...], preferred_element_type=jnp.float32)
    acc = acc + b_ref[...]
    if act == "relu":
        acc = jnp.maximum(acc, 0.0)
    o_ref[...] = acc


def _mm(x, w, b, act="none", blk=MMBLK):
    m, kin = x.shape
    kout = w.shape[1]
    assert m % blk == 0
    grid = (m // blk,)
    return pl.pallas_call(
        functools.partial(_mm_body, act=act),
        grid=grid,
        in_specs=[
            pl.BlockSpec((blk, kin), lambda i: (i, 0)),
            pl.BlockSpec((kin, kout), lambda i: (0, 0)),
            pl.BlockSpec((1, kout), lambda i: (0, 0)),
        ],
        out_specs=pl.BlockSpec((blk, kout), lambda i: (i, 0)),
        out_shape=jax.ShapeDtypeStruct((m, kout), jnp.float32),
    )(x, w, b.reshape(1, kout))


def _gelu(x):
    return 0.5 * x * (1.0 + jax.lax.erf(x * np.float32(1.0 / np.sqrt(2.0))))


def _post_body(agg_ref, h_ref, wa_ref, ba_ref, g_ref, o_ref, *, act):
    g = _gelu(agg_ref[...])
    o = jnp.dot(g, wa_ref[...], preferred_element_type=jnp.float32)
    o = o + ba_ref[...] + g_ref[...] * h_ref[...]
    if act == "relu":
        o = jnp.maximum(o, 0.0)
    o_ref[...] = o


def _post(agg, h, wa, ba, gamma, act="none", blk=MMBLK):
    m, k = agg.shape
    grid = (m // blk,)
    return pl.pallas_call(
        functools.partial(_post_body, act=act),
        grid=grid,
        in_specs=[
            pl.BlockSpec((blk, k), lambda i: (i, 0)),
            pl.BlockSpec((blk, k), lambda i: (i, 0)),
            pl.BlockSpec((k, k), lambda i: (0, 0)),
            pl.BlockSpec((1, k), lambda i: (0, 0)),
            pl.BlockSpec((1, 1), lambda i: (0, 0)),
        ],
        out_specs=pl.BlockSpec((blk, k), lambda i: (i, 0)),
        out_shape=jax.ShapeDtypeStruct((m, k), jnp.float32),
    )(agg, h, wa, ba.reshape(1, k), gamma.reshape(1, 1))


def _dec_body(pg_ref, qg_ref, at_ref, w1c_ref, b1_ref, w2_ref, b2_ref, o_ref):
    s = pg_ref[...] + qg_ref[...] + b1_ref[...]
    s = s + jnp.dot(at_ref[...], w1c_ref[...], preferred_element_type=jnp.float32)
    s = jnp.maximum(s, 0.0)
    o_ref[...] = (jnp.dot(s, w2_ref[...], preferred_element_type=jnp.float32)
                  + b2_ref[...])


def _dec_final(pg, qg, attr, w1c, b1, w2, b2, blk=MMBLK):
    m, k = pg.shape
    ea = attr.shape[1]
    grid = (m // blk,)
    return pl.pallas_call(
        _dec_body,
        grid=grid,
        in_specs=[
            pl.BlockSpec((blk, k), lambda i: (i, 0)),
            pl.BlockSpec((blk, k), lambda i: (i, 0)),
            pl.BlockSpec((blk, ea), lambda i: (i, 0)),
            pl.BlockSpec((ea, k), lambda i: (0, 0)),
            pl.BlockSpec((1, k), lambda i: (0, 0)),
            pl.BlockSpec((k, 1), lambda i: (0, 0)),
            pl.BlockSpec((1, 1), lambda i: (0, 0)),
        ],
        out_specs=pl.BlockSpec((blk, 1), lambda i: (i, 0)),
        out_shape=jax.ShapeDtypeStruct((m, 1), jnp.float32),
    )(pg, qg, attr, w1c, b1.reshape(1, k), w2, b2.reshape(1, 1))


def _fold_rel(w, b, rel, scale=None):
    wf = jnp.einsum("ihd,hde->ihe", w.reshape(128, HEADS, DH), rel)
    bf = jnp.einsum("hd,hde->he", b.reshape(HEADS, DH), rel)
    if scale is not None:
        wf = wf * scale[None, :, None]
        bf = bf * scale[:, None]
    return wf.reshape(128, 128), bf.reshape(128)


def _layer_weights(params, c):
    out = {}
    rel_of_src = {"product": "pw", "warehouse": "wp"}
    for t in ("product", "warehouse"):
        r = rel_of_src[t]
        scale = params[c + "_prel_" + r] * np.float32(1.0 / np.sqrt(DH))
        wk, bk = _fold_rel(params[c + "_k_" + t + "_w"],
                           params[c + "_k_" + t + "_b"],
                           params[c + "_arel_" + r], scale)
        wv, bv = _fold_rel(params[c + "_v_" + t + "_w"],
                           params[c + "_v_" + t + "_b"],
                           params[c + "_mrel_" + r])
        wcat = jnp.concatenate(
            [params[c + "_q_" + t + "_w"], wk, wv], axis=1)
        bcat = jnp.concatenate(
            [params[c + "_q_" + t + "_b"], bk, bv], axis=0)
        out[t] = (wcat, bcat)
    return out


def _edge_pass(k_rel_s, q_d, v_rel_s, src, dst):
    kg, qg, vg = _gather3(k_rel_s, src, q_d, dst, v_rel_s, src)
    alpha = _alpha_tc(kg, qg)
    amax = jax.ops.segment_max(alpha, dst, num_segments=N_NODES)
    amax = jnp.where(jnp.isfinite(amax), amax, 0.0)
    ex = jnp.exp(alpha - amax[dst])
    par = (dst & 1).astype(jnp.float32)[:, None]
    nd = _scatter_nd(_msg128_tc(vg, ex, par), dst >> 1)
    flat = nd.reshape(HEADS, NPAD, 64)[:, :N_NODES, :]
    num = flat[:, :, :DH].transpose(1, 0, 2).reshape(N_NODES, 128)
    den = flat[:, :, DH].T
    return num / (den + 1e-16).repeat(DH, axis=1)


def _hgt_layer(c, h, e_pw, e_wp, params):
    lw = _layer_weights(params, c)
    proj = {}
    for t in ("product", "warehouse"):
        w, b = lw[t]
        z = _mm(h[t], w, b)
        proj[t] = (z[:, :128], z[:, 128:256], z[:, 256:384])
    agg = {}
    for r, s, d, ei in (("pw", "product", "warehouse", e_pw),
                        ("wp", "warehouse", "product", e_wp)):
        agg[d] = _edge_pass(proj[s][1], proj[d][0], proj[s][2], ei[0], ei[1])
    out = {}
    for t in ("product", "warehouse"):
        beta = jax.nn.sigmoid(params[c + "_skip_" + t])
        wa = params[c + "_a_" + t + "_w"] * beta
        ba = params[c + "_a_" + t + "_b"] * beta
        out[t] = _post(agg[t], h[t], wa, ba, 1.0 - beta,
                       act="relu" if c == "c1" else "none")
    return out


def kernel(x_product, x_warehouse, edge_index_pw, edge_index_wp,
           target_edge_index, target_edge_attr, params):
    h = {
        "product": _mm(x_product, params["in_product_w"],
                       params["in_product_b"], act="relu"),
        "warehouse": _mm(x_warehouse, params["in_warehouse_w"],
                         params["in_warehouse_b"], act="relu"),
    }
    h = _hgt_layer("c1", h, edge_index_pw, edge_index_wp, params)
    h = _hgt_layer("c2", h, edge_index_pw, edge_index_wp, params)

    w1 = params["dec1_w"]
    zeros = jnp.zeros((128,), jnp.float32)
    p_arr = _mm(h["product"], w1[:128], zeros)
    q_arr = _mm(h["warehouse"], w1[128:256], zeros)
    src, dst = target_edge_index[0], target_edge_index[1]
    out = _dec_final(p_arr[src], q_arr[dst], target_edge_attr,
                     w1[256:260], params["dec1_b"],
                     params["dec2_w"], params["dec2_b"])
    return out.reshape(-1)
